# Initial kernel scaffold; baseline (speedup 1.0000x reference)
import functools

import jax
import jax.numpy as jnp
from jax import lax
from jax.experimental import pallas as pl
from jax.experimental.pallas import tpu as pltpu
from jax.experimental.pallas import tpu_sc as plsc

B, L, H = 16, 2048, 1024
V = 50000
MAXPOS = 2048
P = 8

NC, NS = 2, 16
NW = NC * NS
LBLK = L // NW
C = 32
NSUB = LBLK // C
HV = H // 16


def _body(ids_hbm, vec_hbm, ipos_hbm, word_hbm, pemb_hbm, out_hbm,
          ids_v, ipos_v, vec_v, pos_v, out_v, sem):
    cid = lax.axis_index("c")
    sid = lax.axis_index("s")
    wid = sid * NC + cid
    lb = wid * LBLK

    pltpu.sync_copy(ipos_hbm, ipos_v)
    lane = lax.broadcasted_iota(jnp.int32, (16,), 0)

    for s in range(NSUB):
        base = lb + s * C
        pltpu.sync_copy(pemb_hbm.at[pl.ds(base + 1, C)], pos_v)

        def seq_body(b, carry):
            pltpu.sync_copy(ids_hbm.at[b, pl.ds(base, C)], ids_v)
            pltpu.async_copy(word_hbm.at[ids_v], out_v, sem).wait()
            pltpu.sync_copy(vec_hbm.at[b], vec_v)

            def row_body(r, c2):
                for k in range(HV):
                    plsc.addupdate(out_v.at[r, pl.ds(k * 16, 16)],
                                   pos_v[r, pl.ds(k * 16, 16)])
                return c2
            lax.fori_loop(0, C, row_body, 0)

            rel = ipos_v[pl.ds(b * P, 16)] - base

            def ins_body(j, c2):
                pj = jnp.max(jnp.where(lane == j, rel, -(2 ** 30)))

                @pl.when((pj >= 0) & (pj < C))
                def _():
                    for k in range(HV):
                        out_v[pj, pl.ds(k * 16, 16)] = (
                            vec_v[pl.ds(k * 16, 16)]
                            + pos_v[pj, pl.ds(k * 16, 16)])
                return c2
            lax.fori_loop(0, P, ins_body, 0)

            pltpu.sync_copy(out_v, out_hbm.at[b, pl.ds(base, C)])
            return carry
        lax.fori_loop(0, B, seq_body, 0)


@jax.jit
def _run(input_ids, vectors, input_pos_flat, word_emb, pos_emb):
    mesh = plsc.VectorSubcoreMesh(core_axis_name="c", subcore_axis_name="s",
                                  num_cores=NC, num_subcores=NS)
    f = pl.kernel(
        _body,
        out_type=jax.ShapeDtypeStruct((B, L, H), jnp.float32),
        mesh=mesh,
        scratch_types=[
            pltpu.VMEM((C,), jnp.int32),
            pltpu.VMEM((B * P + 16,), jnp.int32),
            pltpu.VMEM((H,), jnp.float32),
            pltpu.VMEM((C, H), jnp.float32),
            pltpu.VMEM((C, H), jnp.float32),
            pltpu.SemaphoreType.DMA,
        ],
    )
    return f(input_ids, vectors, input_pos_flat, word_emb, pos_emb)


def kernel(input_ids, vectors, input_pos, word_emb, pos_emb):
    ipos_flat = jnp.pad(input_pos.astype(jnp.int32).reshape(-1), (0, 16))
    return _run(input_ids.astype(jnp.int32), vectors, ipos_flat,
                word_emb, pos_emb)


# speedup vs baseline: 1.6193x; 1.6193x over previous
import functools

import jax
import jax.numpy as jnp
from jax import lax
from jax.experimental import pallas as pl
from jax.experimental.pallas import tpu as pltpu
from jax.experimental.pallas import tpu_sc as plsc

B, L, H = 16, 2048, 1024
V = 50000
MAXPOS = 2048
P = 8

NC, NS = 2, 16
NW = NC * NS
LBLK = L // NW
C = 32
NSUB = LBLK // C
HV = H // 16


def _body(ids_hbm, vec_hbm, ipos_hbm, word_hbm, pemb_hbm, out_hbm,
          ids_v, ipos_v, vec_v, pos_v, out_v, sem):
    cid = lax.axis_index("c")
    sid = lax.axis_index("s")
    wid = sid * NC + cid
    lb = wid * LBLK

    pltpu.sync_copy(ipos_hbm, ipos_v)

    for s in range(NSUB):
        base = lb + s * C
        pltpu.sync_copy(pemb_hbm.at[pl.ds(base, C)], pos_v)

        def seq_body(b, carry):
            pltpu.sync_copy(ids_hbm.at[pl.ds(b * L + base, C)], ids_v)
            pltpu.async_copy(word_hbm.at[ids_v], out_v, sem).wait()
            pltpu.sync_copy(vec_hbm.at[pl.ds(b * H, H)], vec_v)

            def row_body(r, c2):
                for k in range(HV):
                    plsc.addupdate(out_v.at[r, pl.ds(k * 16, 16)],
                                   pos_v[r, pl.ds(k * 16, 16)])
                return c2
            lax.fori_loop(0, C, row_body, 0)

            rel = ipos_v[pl.ds(b * P, 16)] - base
            for j in range(P):
                pj = rel[j]

                @pl.when((pj >= 0) & (pj < C))
                def _():
                    for k in range(HV):
                        out_v[pj, pl.ds(k * 16, 16)] = (
                            vec_v[pl.ds(k * 16, 16)]
                            + pos_v[pj, pl.ds(k * 16, 16)])

            pltpu.sync_copy(out_v, out_hbm.at[b, pl.ds(base, C)])
            return carry
        lax.fori_loop(0, B, seq_body, 0)


@jax.jit
def _run(input_ids, vectors, input_pos_flat, word_emb, pos_emb):
    mesh = plsc.VectorSubcoreMesh(core_axis_name="c", subcore_axis_name="s",
                                  num_cores=NC, num_subcores=NS)
    f = pl.kernel(
        _body,
        out_type=jax.ShapeDtypeStruct((B, L, H), jnp.float32),
        mesh=mesh,
        scratch_types=[
            pltpu.VMEM((C,), jnp.int32),
            pltpu.VMEM((B * P + 16,), jnp.int32),
            pltpu.VMEM((H,), jnp.float32),
            pltpu.VMEM((C, H), jnp.float32),
            pltpu.VMEM((C, H), jnp.float32),
            pltpu.SemaphoreType.DMA,
        ],
    )
    return f(input_ids, vectors, input_pos_flat, word_emb, pos_emb)


def kernel(input_ids, vectors, input_pos, word_emb, pos_emb):
    ipos_flat = jnp.pad(input_pos.astype(jnp.int32).reshape(-1), (0, 16))
    pemb_shift = lax.slice(pos_emb, (1, 0), (L + 1, H))
    return _run(input_ids.astype(jnp.int32).reshape(-1), vectors.reshape(-1),
                ipos_flat, word_emb, pemb_shift)
